# Initial kernel scaffold; baseline (speedup 1.0000x reference)
import jax
import jax.numpy as jnp
from jax.experimental import pallas as pl


def kernel(env_indices, step_count, slot_card_rows, slot_occupied, slot_tapped, game_info, trace_kind_id, pending_kind_id, option_kind_ids, option_scalars, option_mask, option_ref_slot_idx, option_ref_card_row, target_mask, target_type_ids, target_scalars, target_overflow, target_ref_slot_idx, target_ref_is_player, target_ref_is_self, may_selected, old_log_probs, values, perspective_player_idx):
    raise NotImplementedError("write your pallas kernel here")


# speedup vs baseline: 2.2840x; 2.2840x over previous
import jax
import jax.numpy as jnp
from jax import lax
from jax.experimental import pallas as pl
from jax.experimental.pallas import tpu as pltpu

E = 256
T = 64
EB = 8
GRID = E // EB


def _body(step_sref, *refs):
    n_in = 18
    ins = refs[:n_in]
    outs = refs[n_in:]
    (in_scr, in_socc, in_stap, in_gi, in_okid, in_oscal, in_omask,
     in_oslot, in_ocard, in_tmask, in_ttype, in_tscal, in_tovf,
     in_tslot, in_ispl, in_iself, in_scal6, in_steprow) = ins
    (o_scr, o_socc, o_stap, o_gi, o_okid, o_oscal, o_omask,
     o_oslot, o_ocard, o_tmask, o_ttype, o_tscal, o_tovf,
     o_tslot, o_ispl, o_iself, o_scal6, o_newstep) = outs

    i = pl.program_id(0)

    three_d = (
        (o_scr, in_scr), (o_socc, in_socc), (o_stap, in_stap),
        (o_gi, in_gi), (o_okid, in_okid), (o_oscal, in_oscal),
        (o_omask, in_omask), (o_oslot, in_oslot), (o_ocard, in_ocard),
        (o_tmask, in_tmask), (o_ttype, in_ttype), (o_tscal, in_tscal),
        (o_tovf, in_tovf), (o_tslot, in_tslot), (o_ispl, in_ispl),
        (o_iself, in_iself), (o_scal6, in_scal6),
    )
    for o, _ in three_d:
        o[...] = jnp.zeros_like(o)

    for j in range(EB):
        s = step_sref[i * EB + j]
        for o, x in three_d:
            rest = o.shape[2]
            o[pl.ds(j, 1), pl.ds(s, 1), :] = x[pl.ds(j, 1), :].reshape(1, 1, rest)

    @pl.when(i == 0)
    def _():
        o_newstep[...] = in_steprow[...] + 1


def kernel(env_indices, step_count, slot_card_rows, slot_occupied, slot_tapped,
           game_info, trace_kind_id, pending_kind_id, option_kind_ids,
           option_scalars, option_mask, option_ref_slot_idx, option_ref_card_row,
           target_mask, target_type_ids, target_scalars, target_overflow,
           target_ref_slot_idx, target_ref_is_player, target_ref_is_self,
           may_selected, old_log_probs, values, perspective_player_idx):
    B = E
    Z = slot_card_rows.shape[1]
    GID = game_info.shape[1]
    O = option_kind_ids.shape[1]
    OSD = option_scalars.shape[2]
    TG = target_mask.shape[2]
    TSD = target_scalars.shape[3]

    def pack_bool(x):
        return lax.bitcast_convert_type(x.astype(jnp.uint8), jnp.uint32)

    ispl = pack_bool(target_ref_is_player)
    iself = pack_bool(target_ref_is_self)

    as_f32 = lambda v: lax.bitcast_convert_type(v, jnp.float32)
    scal6 = jnp.stack(
        [as_f32(trace_kind_id), as_f32(pending_kind_id), may_selected,
         old_log_probs, values, as_f32(perspective_player_idx)], axis=1)

    flat_ins = [
        slot_card_rows, slot_occupied, slot_tapped, game_info,
        option_kind_ids, option_scalars.reshape(B, O * OSD), option_mask,
        option_ref_slot_idx, option_ref_card_row,
        target_mask.reshape(B, O * TG), target_type_ids.reshape(B, O * TG),
        target_scalars.reshape(B, O * TG * TSD), target_overflow,
        target_ref_slot_idx.reshape(B, O * TG), ispl, iself, scal6,
        step_count.reshape(1, E),
    ]

    rests = [Z, Z, Z, GID, O, O * OSD, O, O, O, O * TG, O * TG,
             O * TG * TSD, O, O * TG, O, O, 6]
    dtypes = [jnp.int32, jnp.float32, jnp.float32, jnp.float32,
              jnp.int32, jnp.float32, jnp.float32, jnp.int32, jnp.int32,
              jnp.float32, jnp.int32, jnp.float32, jnp.float32, jnp.int32,
              jnp.uint32, jnp.uint32, jnp.float32]

    out_shapes = [jax.ShapeDtypeStruct((E, T, r), d) for r, d in zip(rests, dtypes)]
    out_shapes.append(jax.ShapeDtypeStruct((1, E), jnp.int32))

    in_specs = [pl.BlockSpec((EB, r), lambda i, *_: (i, 0)) for r in rests]
    in_specs.append(pl.BlockSpec((1, E), lambda i, *_: (0, 0)))
    out_specs = [pl.BlockSpec((EB, T, r), lambda i, *_: (i, 0, 0)) for r in rests]
    out_specs.append(pl.BlockSpec((1, E), lambda i, *_: (0, 0)))

    grid_spec = pltpu.PrefetchScalarGridSpec(
        num_scalar_prefetch=1,
        grid=(GRID,),
        in_specs=in_specs,
        out_specs=out_specs,
    )

    outs = pl.pallas_call(
        _body,
        grid_spec=grid_spec,
        out_shape=out_shapes,
        compiler_params=pltpu.CompilerParams(
            dimension_semantics=("arbitrary",),
        ),
    )(step_count, *flat_ins)

    (b_scr, b_socc, b_stap, b_gi, b_okid, b_oscal, b_omask, b_oslot,
     b_ocard, b_tmask, b_ttype, b_tscal, b_tovf, b_tslot, b_ispl,
     b_iself, b_scal6, b_newstep) = outs

    def unpack_bool(x):
        return lax.bitcast_convert_type(x, jnp.uint8).astype(jnp.bool_)

    as_i32 = lambda v: lax.bitcast_convert_type(v, jnp.int32)

    return (
        b_scr, b_socc, b_stap, b_gi,
        as_i32(b_scal6[:, :, 0]), as_i32(b_scal6[:, :, 1]),
        b_okid, b_oscal.reshape(E, T, O, OSD), b_omask, b_oslot, b_ocard,
        b_tmask.reshape(E, T, O, TG), b_ttype.reshape(E, T, O, TG),
        b_tscal.reshape(E, T, O, TG, TSD), b_tovf,
        b_tslot.reshape(E, T, O, TG), unpack_bool(b_ispl), unpack_bool(b_iself),
        b_scal6[:, :, 2], b_scal6[:, :, 3], b_scal6[:, :, 4],
        as_i32(b_scal6[:, :, 5]),
        b_newstep.reshape(E),
    )
